# baseline (speedup 1.0000x reference)
import functools

import jax
import jax.numpy as jnp
from jax import lax
from jax.experimental import pallas as pl
from jax.experimental.pallas import tpu as pltpu
from jax.experimental.pallas import tpu_sc as plsc

CHUNK = 32
NBUF = 2


def _make_sc_gather(V, D, B):
    info = plsc.get_sparse_core_info()
    NW = info.num_cores * info.num_subcores
    NC = info.num_cores
    b_per_w = B // NW
    n_chunks = b_per_w // CHUNK
    n_outer = n_chunks // NBUF
    mesh = plsc.VectorSubcoreMesh(core_axis_name="c", subcore_axis_name="s")

    @functools.partial(
        pl.kernel,
        mesh=mesh,
        out_type=jax.ShapeDtypeStruct((B, D), jnp.float32),
        scratch_types=[
            pltpu.VMEM((n_chunks, CHUNK), jnp.int32),
            *[pltpu.VMEM((CHUNK, D), jnp.float32) for _ in range(NBUF)],
            *[pltpu.SemaphoreType.DMA for _ in range(2 * NBUF)],
        ],
    )
    def k(table_hbm, idx_hbm, out_hbm, idx_v, *rest):
        bufs = rest[:NBUF]
        gsems = rest[NBUF:2 * NBUF]
        ssems = rest[2 * NBUF:]
        wid = lax.axis_index("s") * NC + lax.axis_index("c")
        base = wid * b_per_w
        pltpu.sync_copy(idx_hbm.at[wid], idx_v)

        def wait_gather(b, c):
            pltpu.make_async_copy(
                table_hbm.at[idx_v.at[c]], bufs[b], gsems[b]).wait()

        def start_store(b, c):
            pltpu.async_copy(
                bufs[b], out_hbm.at[pl.ds(base + c * CHUNK, CHUNK)], ssems[b])

        def wait_store(b, c):
            pltpu.make_async_copy(
                bufs[b], out_hbm.at[pl.ds(base + c * CHUNK, CHUNK)],
                ssems[b]).wait()

        for b in range(NBUF):
            pltpu.async_copy(table_hbm.at[idx_v.at[b]], bufs[b], gsems[b])

        def outer_body(i, carry):
            c0 = i * NBUF
            for b in range(NBUF):
                wait_gather(b, c0 + b)
                start_store(b, c0 + b)
            for b in range(NBUF):
                wait_store(b, c0 + b)
                pltpu.async_copy(
                    table_hbm.at[idx_v.at[c0 + b + NBUF]], bufs[b], gsems[b])
            return carry

        lax.fori_loop(0, n_outer - 1, outer_body, 0, unroll=False)

        c0 = (n_outer - 1) * NBUF
        for b in range(NBUF):
            wait_gather(b, c0 + b)
            start_store(b, c0 + b)
        for b in range(NBUF):
            wait_store(b, c0 + b)

    return k


def kernel(pos, wpe):
    B_, S_ = pos.shape
    V, D = wpe.shape
    flat = pos.reshape(-1).astype(jnp.int32)
    info = plsc.get_sparse_core_info()
    NW = info.num_cores * info.num_subcores
    idx3 = flat.reshape(NW, -1, CHUNK)
    out = _make_sc_gather(V, D, flat.shape[0])(wpe, idx3)
    return out.reshape(B_, S_, D)


# speedup vs baseline: 1.0276x; 1.0276x over previous
import functools

import jax
import jax.numpy as jnp
from jax import lax
from jax.experimental import pallas as pl
from jax.experimental.pallas import tpu as pltpu
from jax.experimental.pallas import tpu_sc as plsc

CHUNK = 8
NBUF = 8


def _make_sc_gather(V, D, B):
    info = plsc.get_sparse_core_info()
    NW = info.num_cores * info.num_subcores
    NC = info.num_cores
    b_per_w = B // NW
    n_chunks = b_per_w // CHUNK
    n_outer = n_chunks // NBUF
    mesh = plsc.VectorSubcoreMesh(core_axis_name="c", subcore_axis_name="s")

    @functools.partial(
        pl.kernel,
        mesh=mesh,
        out_type=jax.ShapeDtypeStruct((B, D), jnp.float32),
        scratch_types=[
            pltpu.VMEM((n_chunks, CHUNK), jnp.int32),
            *[pltpu.VMEM((CHUNK, D), jnp.float32) for _ in range(NBUF)],
            *[pltpu.SemaphoreType.DMA for _ in range(2 * NBUF)],
        ],
    )
    def k(table_hbm, idx_hbm, out_hbm, idx_v, *rest):
        bufs = rest[:NBUF]
        gsems = rest[NBUF:2 * NBUF]
        ssems = rest[2 * NBUF:]
        wid = lax.axis_index("s") * NC + lax.axis_index("c")
        base = wid * b_per_w
        pltpu.sync_copy(idx_hbm.at[wid], idx_v)

        def wait_gather(b, c):
            pltpu.make_async_copy(
                table_hbm.at[idx_v.at[c]], bufs[b], gsems[b]).wait()

        def start_store(b, c):
            pltpu.async_copy(
                bufs[b], out_hbm.at[pl.ds(base + c * CHUNK, CHUNK)], ssems[b])

        def wait_store(b, c):
            pltpu.make_async_copy(
                bufs[b], out_hbm.at[pl.ds(base + c * CHUNK, CHUNK)],
                ssems[b]).wait()

        for b in range(NBUF):
            pltpu.async_copy(table_hbm.at[idx_v.at[b]], bufs[b], gsems[b])

        def outer_body(i, carry):
            c0 = i * NBUF
            for b in range(NBUF):
                wait_gather(b, c0 + b)
                start_store(b, c0 + b)
            for b in range(NBUF):
                wait_store(b, c0 + b)
                pltpu.async_copy(
                    table_hbm.at[idx_v.at[c0 + b + NBUF]], bufs[b], gsems[b])
            return carry

        lax.fori_loop(0, n_outer - 1, outer_body, 0, unroll=False)

        c0 = (n_outer - 1) * NBUF
        for b in range(NBUF):
            wait_gather(b, c0 + b)
            start_store(b, c0 + b)
        for b in range(NBUF):
            wait_store(b, c0 + b)

    return k


def kernel(pos, wpe):
    B_, S_ = pos.shape
    V, D = wpe.shape
    flat = pos.reshape(-1).astype(jnp.int32)
    info = plsc.get_sparse_core_info()
    NW = info.num_cores * info.num_subcores
    idx3 = flat.reshape(NW, -1, CHUNK)
    out = _make_sc_gather(V, D, flat.shape[0])(wpe, idx3)
    return out.reshape(B_, S_, D)
